# baseline (speedup 1.0000x reference)
import jax
import jax.numpy as jnp
from jax import lax
from jax.experimental import pallas as pl
from jax.experimental.pallas import tpu as pltpu
from jax.experimental.pallas import tpu_sc as plsc

D = 128
N = 100000
NC, NS = 2, 16
NW = NC * NS
CHUNK = 128
CPW = 25
BPW = CHUNK * CPW
NPAD = NW * BPW


def _emb_body(z_hbm, tab_hbm, out_hbm, idx_v, buf0, buf1, sem0, sem1):
    wid = lax.axis_index("s") * NC + lax.axis_index("c")
    base = wid * BPW
    pltpu.sync_copy(z_hbm.at[pl.ds(base, BPW)], idx_v)
    bufs = (buf0, buf1)
    sems = (sem0, sem1)

    def gather(j):
        idx = idx_v.at[pl.ds(j * CHUNK, CHUNK)]
        return pltpu.make_async_copy(tab_hbm.at[idx], bufs[j % 2], sems[j % 2])

    gather(0).start()
    for j in range(CPW):
        if j + 1 < CPW:
            gather(j + 1).start()
        gather(j).wait()
        pltpu.sync_copy(bufs[j % 2], out_hbm.at[pl.ds(base + j * CHUNK, CHUNK)])


@jax.jit
def _emb(z1d, tab):
    f = pl.kernel(
        _emb_body,
        out_type=jax.ShapeDtypeStruct((NPAD, D), jnp.float32),
        mesh=plsc.VectorSubcoreMesh(core_axis_name="c", subcore_axis_name="s"),
        scratch_types=[
            pltpu.VMEM((BPW,), jnp.int32),
            pltpu.VMEM((CHUNK, D), jnp.float32),
            pltpu.VMEM((CHUNK, D), jnp.float32),
            pltpu.SemaphoreType.DMA,
            pltpu.SemaphoreType.DMA,
        ],
    )
    return f(z1d, tab)


def kernel(Z, emb_weight):
    z = Z.astype(jnp.int32)
    z = jnp.pad(z, (0, NPAD - N))
    out = _emb(z, emb_weight)
    return out[:N]


# speedup vs baseline: 1.0016x; 1.0016x over previous
import jax
import jax.numpy as jnp
from jax import lax
from jax.experimental import pallas as pl
from jax.experimental.pallas import tpu as pltpu
from jax.experimental.pallas import tpu_sc as plsc

D = 128
N = 100000
NC, NS = 2, 16
NW = NC * NS
CHUNK = 128
CPW = 25
BPW = CHUNK * CPW
NPAD = NW * BPW


NBUF = 6
LOOK = 3


def _emb_body(z_hbm, tab_hbm, out_hbm, idx_v, bufs, gsems, wsems):
    wid = lax.axis_index("s") * NC + lax.axis_index("c")
    base = wid * BPW
    pltpu.sync_copy(z_hbm.at[pl.ds(base, BPW)], idx_v)

    def gather(j):
        b = j % NBUF
        idx = idx_v.at[pl.ds(j * CHUNK, CHUNK)]
        return pltpu.make_async_copy(tab_hbm.at[idx], bufs.at[b], gsems.at[b])

    def write(j):
        b = j % NBUF
        return pltpu.make_async_copy(
            bufs.at[b], out_hbm.at[pl.ds(base + j * CHUNK, CHUNK)], wsems.at[b])

    for j in range(min(LOOK, CPW)):
        gather(j).start()
    for j in range(CPW):
        gather(j).wait()
        write(j).start()
        nj = j + LOOK
        if nj < CPW:
            if nj >= NBUF:
                write(nj - NBUF).wait()
            gather(nj).start()
    for j in range(max(0, CPW - NBUF), CPW):
        write(j).wait()


@jax.jit
def _emb(z1d, tab):
    f = pl.kernel(
        _emb_body,
        out_type=jax.ShapeDtypeStruct((NPAD, D), jnp.float32),
        mesh=plsc.VectorSubcoreMesh(core_axis_name="c", subcore_axis_name="s"),
        scratch_types=[
            pltpu.VMEM((BPW,), jnp.int32),
            pltpu.VMEM((NBUF, CHUNK, D), jnp.float32),
            pltpu.SemaphoreType.DMA((NBUF,)),
            pltpu.SemaphoreType.DMA((NBUF,)),
        ],
    )
    return f(z1d, tab)


def kernel(Z, emb_weight):
    z = Z.astype(jnp.int32)
    z = jnp.pad(z, (0, NPAD - N))
    out = _emb(z, emb_weight)
    return out[:N]


# speedup vs baseline: 2.0733x; 2.0700x over previous
import jax
import jax.numpy as jnp
from jax import lax
from jax.experimental import pallas as pl
from jax.experimental.pallas import tpu as pltpu
from jax.experimental.pallas import tpu_sc as plsc

D = 128
NROWS = 100
N = 100000
NC, NS = 2, 16
NW = NC * NS
CHUNK = 128
CPW = 25
BPW = CHUNK * CPW
NPAD = NW * BPW
NBUF = 5
NOUT = CPW // NBUF
NLANE = 16


def _emb_body(z_hbm, tab_hbm, out_hbm, tab_v, idx_v, b0, b1, b2, b3, b4, wsems):
    bufs = (b0, b1, b2, b3, b4)
    wid = lax.axis_index("s") * NC + lax.axis_index("c")
    base = wid * BPW
    pltpu.sync_copy(tab_hbm, tab_v)
    pltpu.sync_copy(z_hbm.at[pl.ds(base, BPW)], idx_v)

    def write(j, b):
        return pltpu.make_async_copy(
            bufs[b], out_hbm.at[pl.ds((base + j * CHUNK) * D, CHUNK * D)],
            wsems.at[b])

    def outer(it, carry):
        for b in range(NBUF):
            j = it * NBUF + b

            @pl.when(it > 0)
            def _():
                write(j - NBUF, b).wait()

            @plsc.parallel_loop(0, CHUNK, step=NLANE)
            def _group(i):
                zv = idx_v[pl.ds(j * CHUNK + i, NLANE)]
                for k in range(NLANE):
                    off = zv[k] * D
                    dst = (i + k) * D
                    for c in range(D // NLANE):
                        bufs[b][pl.ds(dst + c * NLANE, NLANE)] = (
                            tab_v[pl.ds(off + c * NLANE, NLANE)])

            write(j, b).start()
        return carry

    lax.fori_loop(0, NOUT, outer, 0)
    for b in range(NBUF):
        write((NOUT - 1) * NBUF + b, b).wait()


@jax.jit
def _emb(z1d, tab_flat):
    f = pl.kernel(
        _emb_body,
        out_type=jax.ShapeDtypeStruct((NPAD * D,), jnp.float32),
        mesh=plsc.VectorSubcoreMesh(core_axis_name="c", subcore_axis_name="s"),
        scratch_types=[
            pltpu.VMEM((NROWS * D,), jnp.float32),
            pltpu.VMEM((BPW,), jnp.int32),
        ] + [pltpu.VMEM((CHUNK * D,), jnp.float32) for _ in range(NBUF)] + [
            pltpu.SemaphoreType.DMA((NBUF,)),
        ],
    )
    return f(z1d, tab_flat)


def kernel(Z, emb_weight):
    z = Z.astype(jnp.int32)
    z = jnp.pad(z, (0, NPAD - N))
    out = _emb(z, emb_weight.reshape(-1))
    return out.reshape(NPAD, D)[:N]


# speedup vs baseline: 2.5776x; 1.2432x over previous
import jax
import jax.numpy as jnp
from jax import lax
from jax.experimental import pallas as pl
from jax.experimental.pallas import tpu as pltpu
from jax.experimental.pallas import tpu_sc as plsc

D = 128
NROWS = 100
N = 100000
NC, NS = 2, 16
NW = NC * NS
BPW = N // NW
CHUNK = 125
CPW = BPW // CHUNK
NBUF = 5
NOUT = CPW // NBUF
NLANE = 16
GRP = (CHUNK // NLANE) * NLANE
TAIL = CHUNK - GRP
ISTAGE = BPW + 11
ZPAD = 8


def _emb_body(z_hbm, tab_hbm, out_hbm, tab_v, idx_v, b0, b1, b2, b3, b4, wsems):
    bufs = (b0, b1, b2, b3, b4)
    wid = lax.axis_index("s") * NC + lax.axis_index("c")
    base = wid * BPW
    astart = (base // 8) * 8
    s = base - astart
    pltpu.sync_copy(tab_hbm, tab_v)
    pltpu.sync_copy(z_hbm.at[pl.ds(astart, ISTAGE)], idx_v)

    def write(j, b):
        return pltpu.make_async_copy(
            bufs[b], out_hbm.at[pl.ds((base + j * CHUNK) * D, CHUNK * D)],
            wsems.at[b])

    def assemble(zv, buf, dst0, nk):
        for k in range(nk):
            off = zv[k] * D
            dst = dst0 + k * D
            for c in range(D // NLANE):
                buf[pl.ds(dst + c * NLANE, NLANE)] = (
                    tab_v[pl.ds(off + c * NLANE, NLANE)])

    def outer(it, carry):
        for b in range(NBUF):
            j = it * NBUF + b

            @pl.when(it > 0)
            def _():
                write(j - NBUF, b).wait()

            @plsc.parallel_loop(0, GRP, step=NLANE)
            def _group(i):
                zv = idx_v[pl.ds(s + j * CHUNK + i, NLANE)]
                assemble(zv, bufs[b], i * D, NLANE)

            zt = idx_v[pl.ds(s + j * CHUNK + GRP, NLANE)]
            assemble(zt, bufs[b], GRP * D, TAIL)
            write(j, b).start()
        return carry

    lax.fori_loop(0, NOUT, outer, 0)
    for b in range(NBUF):
        write((NOUT - 1) * NBUF + b, b).wait()


@jax.jit
def _emb(z1d, tab_flat):
    f = pl.kernel(
        _emb_body,
        out_type=jax.ShapeDtypeStruct((N * D,), jnp.float32),
        mesh=plsc.VectorSubcoreMesh(core_axis_name="c", subcore_axis_name="s"),
        scratch_types=[
            pltpu.VMEM((NROWS * D,), jnp.float32),
            pltpu.VMEM((ISTAGE,), jnp.int32),
        ] + [pltpu.VMEM((CHUNK * D,), jnp.float32) for _ in range(NBUF)] + [
            pltpu.SemaphoreType.DMA((NBUF,)),
        ],
    )
    return f(z1d, tab_flat)


def kernel(Z, emb_weight):
    z = jnp.pad(Z.astype(jnp.int32), (0, ZPAD))
    out = _emb(z, emb_weight.reshape(-1))
    return out.reshape(N, D)
